# baseline (speedup 1.0000x reference)
import functools

import jax
import jax.numpy as jnp
from jax import lax
from jax.experimental import pallas as pl
from jax.experimental.pallas import tpu as pltpu
from jax.experimental.pallas import tpu_sc as plsc

NUM_AP = 5000
U = 10
NUM_UE = 50000
A = 16
E = 1600000
M2 = 16
H = 64

NC = 2
NS = 16
NW = NC * NS
EPT = E // NW
CH = 1000
STRIPE = 3128
SEG_PAD = NS * STRIPE

def _mesh():
    return plsc.VectorSubcoreMesh(core_axis_name="c", subcore_axis_name="s",
                                  num_cores=NC, num_subcores=NS)


def _sc_gather(table, idx, width):
    ch = 1000 if width == 2 * A else 2000
    nch = EPT // ch

    @functools.partial(
        pl.kernel,
        out_type=jax.ShapeDtypeStruct((E, width), jnp.float32),
        mesh=_mesh(),
        compiler_params=pltpu.CompilerParams(use_tc_tiling_on_sc=False),
        scratch_types=[
            pltpu.VMEM((EPT,), jnp.int32),
            pltpu.VMEM((2, ch, width), jnp.float32),
            pltpu.SemaphoreType.DMA,
            pltpu.SemaphoreType.DMA,
        ],
    )
    def k(table_hbm, idx_hbm, out_hbm, idx_v, rows_v, sem0, sem1):
        wid = lax.axis_index("c") * NS + lax.axis_index("s")
        base = wid * EPT
        pltpu.sync_copy(idx_hbm.at[pl.ds(base, EPT)], idx_v)

        sems = [sem0, sem1]
        descs = [None] * nch

        def start(j):
            descs[j] = pltpu.async_copy(
                table_hbm.at[idx_v.at[pl.ds(j * ch, ch)]],
                rows_v.at[j % 2], sems[j % 2])

        start(0)
        for j in range(nch):
            if j + 1 < nch:
                start(j + 1)
            descs[j].wait()
            pltpu.sync_copy(rows_v.at[j % 2], out_hbm.at[pl.ds(base + j * ch, ch)])

    return k(table, idx)


def _sc_segment_sum(rows, idx, zstripe):
    ch = 2000
    nch = EPT // ch

    @functools.partial(
        pl.kernel,
        out_type=[
            jax.ShapeDtypeStruct((SEG_PAD, A), jnp.float32),
            jax.ShapeDtypeStruct((SEG_PAD, A), jnp.float32),
        ],
        mesh=_mesh(),
        compiler_params=pltpu.CompilerParams(use_tc_tiling_on_sc=False),
        scratch_types=[
            pltpu.VMEM_SHARED((SEG_PAD, A), jnp.float32),
            pltpu.VMEM((2, ch), jnp.int32),
            pltpu.VMEM((2, ch, A), jnp.float32),
            pltpu.SemaphoreType.DMA,
            pltpu.SemaphoreType.DMA,
            pltpu.SemaphoreType.DMA,
            pltpu.SemaphoreType.DMA,
        ],
    )
    def k(rows_hbm, idx_hbm, z_hbm, out0, out1, acc, idx_v, rows_v,
          semr0, semr1, semi0, semi1):
        cid = lax.axis_index("c")
        sid = lax.axis_index("s")
        wid = cid * NS + sid
        base = wid * EPT
        pltpu.sync_copy(z_hbm, acc.at[pl.ds(sid * STRIPE, STRIPE)])
        plsc.subcore_barrier()

        semr = [semr0, semr1]
        semi = [semi0, semi1]
        dr = [None] * nch
        di = [None] * nch

        def start(j):
            b = j % 2
            dr[j] = pltpu.async_copy(rows_hbm.at[pl.ds(base + j * ch, ch)],
                                     rows_v.at[b], semr[b])
            di[j] = pltpu.async_copy(idx_hbm.at[pl.ds(base + j * ch, ch)],
                                     idx_v.at[b], semi[b])

        start(0)
        for j in range(nch):
            if j + 1 < nch:
                start(j + 1)
            dr[j].wait()
            di[j].wait()
            pltpu.sync_copy(rows_v.at[j % 2], acc.at[idx_v.at[j % 2]], add=True)

        plsc.subcore_barrier()

        @pl.when(cid == 0)
        def _():
            pltpu.sync_copy(acc.at[pl.ds(sid * STRIPE, STRIPE)],
                            out0.at[pl.ds(sid * STRIPE, STRIPE)])

        @pl.when(cid == 1)
        def _():
            pltpu.sync_copy(acc.at[pl.ds(sid * STRIPE, STRIPE)],
                            out1.at[pl.ds(sid * STRIPE, STRIPE)])

    return k(rows, idx, zstripe)


def _sum_over_u_kernel(pr_ref, pi_ref, out_ref):
    s2r = jnp.sum(pr_ref[...], axis=1)
    s2i = jnp.sum(pi_ref[...], axis=1)
    out_ref[...] = jnp.concatenate([s2r, s2i], axis=1)


def _tc_s2_table(p_real, p_imag):
    BA = 200
    return pl.pallas_call(
        _sum_over_u_kernel,
        grid=(NUM_AP // BA,),
        in_specs=[
            pl.BlockSpec((BA, U, A), lambda i: (i, 0, 0)),
            pl.BlockSpec((BA, U, A), lambda i: (i, 0, 0)),
        ],
        out_specs=pl.BlockSpec((BA, 2 * A), lambda i: (i, 0)),
        out_shape=jax.ShapeDtypeStruct((NUM_AP, 2 * A), jnp.float32),
    )(p_real, p_imag)


def _edge_mlp_kernel(plr_ref, pli_ref, g_ref, w2a_ref,
                     b2a_ref, w2b_ref, b2b_ref, msg_ref):
    plr = plr_ref[...]
    pli = pli_ref[...]
    g = g_ref[...]
    feat = jnp.concatenate([plr, g[:, :A], pli, g[:, A:]], axis=1)
    h = jnp.dot(feat, w2a_ref[...], preferred_element_type=jnp.float32) + b2a_ref[...]
    h = jnp.maximum(h, 0.0)
    msg_ref[...] = (jnp.dot(h, w2b_ref[...], preferred_element_type=jnp.float32)
                    + b2b_ref[...])


def _tc_edge_mlp(plint_real, plint_imag, g, w2a, b2a, w2b, b2b):
    B = 8000
    return pl.pallas_call(
        _edge_mlp_kernel,
        grid=(E // B,),
        in_specs=[
            pl.BlockSpec((B, A), lambda i: (i, 0)),
            pl.BlockSpec((B, A), lambda i: (i, 0)),
            pl.BlockSpec((B, 2 * A), lambda i: (i, 0)),
            pl.BlockSpec((4 * A, H), lambda i: (0, 0)),
            pl.BlockSpec((1, H), lambda i: (0, 0)),
            pl.BlockSpec((H, M2), lambda i: (0, 0)),
            pl.BlockSpec((1, M2), lambda i: (0, 0)),
        ],
        out_specs=pl.BlockSpec((B, M2), lambda i: (i, 0)),
        out_shape=jax.ShapeDtypeStruct((E, M2), jnp.float32),
    )(plint_real, plint_imag, g,
      w2a, b2a.reshape(1, H), w2b, b2b.reshape(1, M2))


def _inner_infer_kernel(pr_ref, pi_ref, qr_ref, qi_ref, out_ref):
    pr = pr_ref[...]
    pi = pi_ref[...]
    qr = qr_ref[...]
    qi = qi_ref[...]
    BA = pr.shape[0]
    vidx = lax.broadcasted_iota(jnp.int32, (BA, U), 1)
    acc = jnp.zeros((BA, U), jnp.float32)
    diag = jnp.zeros((BA, U), jnp.float32)
    for u in range(U):
        pru = pr[:, u:u + 1, :]
        piu = pi[:, u:u + 1, :]
        ir = jnp.sum(pru * qr + piu * qi, axis=-1)
        ii = jnp.sum(pru * qi - piu * qr, axis=-1)
        nrm = ir * ir + ii * ii
        acc = acc + nrm
        diag = diag + jnp.where(vidx == u, nrm, 0.0)
    out_ref[...] = acc - diag


def _tc_inner_infer(p_real, p_imag, pldl_r3, pldl_i3):
    BA = 200
    return pl.pallas_call(
        _inner_infer_kernel,
        grid=(NUM_AP // BA,),
        in_specs=[
            pl.BlockSpec((BA, U, A), lambda i: (i, 0, 0)),
            pl.BlockSpec((BA, U, A), lambda i: (i, 0, 0)),
            pl.BlockSpec((BA, U, A), lambda i: (i, 0, 0)),
            pl.BlockSpec((BA, U, A), lambda i: (i, 0, 0)),
        ],
        out_specs=pl.BlockSpec((BA, U), lambda i: (i, 0)),
        out_shape=jax.ShapeDtypeStruct((NUM_AP, U), jnp.float32),
    )(p_real, p_imag, pldl_r3, pldl_i3)


def _ue_mlp_kernel(inner_ref, plr_ref, pli_ref, m0_ref, m1_ref,
                   w_in_ref, w_plr_ref, w_pli_ref, w_m_ref, b1a_ref,
                   w1b_ref, b1b_ref, uer_ref, uei_ref):
    inner = inner_ref[...]
    m = m0_ref[...] + m1_ref[...]
    h = (inner * w_in_ref[...]
         + jnp.dot(plr_ref[...], w_plr_ref[...], preferred_element_type=jnp.float32)
         + jnp.dot(pli_ref[...], w_pli_ref[...], preferred_element_type=jnp.float32)
         + jnp.dot(m, w_m_ref[...], preferred_element_type=jnp.float32)
         + b1a_ref[...])
    h = jnp.maximum(h, 0.0)
    o = jnp.dot(h, w1b_ref[...], preferred_element_type=jnp.float32) + b1b_ref[...]
    uer_ref[...] = o[:, :A]
    uei_ref[...] = o[:, A:2 * A]


def _tc_ue_mlp(inner, pldl_real, pldl_imag, m0, m1, w1a, b1a, w1b, b1b):
    B = 5000
    return pl.pallas_call(
        _ue_mlp_kernel,
        grid=(NUM_UE // B,),
        in_specs=[
            pl.BlockSpec((B, 1), lambda i: (i, 0)),
            pl.BlockSpec((B, A), lambda i: (i, 0)),
            pl.BlockSpec((B, A), lambda i: (i, 0)),
            pl.BlockSpec((B, M2), lambda i: (i, 0)),
            pl.BlockSpec((B, M2), lambda i: (i, 0)),
            pl.BlockSpec((1, H), lambda i: (0, 0)),
            pl.BlockSpec((A, H), lambda i: (0, 0)),
            pl.BlockSpec((A, H), lambda i: (0, 0)),
            pl.BlockSpec((M2, H), lambda i: (0, 0)),
            pl.BlockSpec((1, H), lambda i: (0, 0)),
            pl.BlockSpec((H, 2 * A), lambda i: (0, 0)),
            pl.BlockSpec((1, 2 * A), lambda i: (0, 0)),
        ],
        out_specs=[
            pl.BlockSpec((B, A), lambda i: (i, 0)),
            pl.BlockSpec((B, A), lambda i: (i, 0)),
        ],
        out_shape=[
            jax.ShapeDtypeStruct((NUM_UE, A), jnp.float32),
            jax.ShapeDtypeStruct((NUM_UE, A), jnp.float32),
        ],
    )(inner, pldl_real, pldl_imag, m0, m1,
      w1a[0:1], w1a[1:1 + A], w1a[1 + A:1 + 2 * A], w1a[1 + 2 * A:1 + 2 * A + M2],
      b1a.reshape(1, H), w1b, b1b.reshape(1, 2 * A))


def _normalize_kernel(r_ref, i_ref, outr_ref, outi_ref, t_ref):
    r = r_ref[...]
    im = i_ref[...]
    mag = jnp.sqrt(r * r + im * im)
    s = jnp.sum(jnp.sum(mag, axis=2, keepdims=True), axis=1, keepdims=True)
    nr = r / s
    ni = im / s
    outr_ref[...] = nr
    outi_ref[...] = ni
    sr = jnp.sum(nr, axis=1)
    si = jnp.sum(ni, axis=1)
    t_ref[...] = sr * sr + si * si


def _tc_normalize(uer3, uei3):
    BA = 200
    return pl.pallas_call(
        _normalize_kernel,
        grid=(NUM_AP // BA,),
        in_specs=[
            pl.BlockSpec((BA, U, A), lambda i: (i, 0, 0)),
            pl.BlockSpec((BA, U, A), lambda i: (i, 0, 0)),
        ],
        out_specs=[
            pl.BlockSpec((BA, U, A), lambda i: (i, 0, 0)),
            pl.BlockSpec((BA, U, A), lambda i: (i, 0, 0)),
            pl.BlockSpec((BA, A), lambda i: (i, 0)),
        ],
        out_shape=[
            jax.ShapeDtypeStruct((NUM_AP, U, A), jnp.float32),
            jax.ShapeDtypeStruct((NUM_AP, U, A), jnp.float32),
            jax.ShapeDtypeStruct((NUM_AP, A), jnp.float32),
        ],
    )(uer3, uei3)


def _prod_kernel(plr_ref, pli_ref, tg_ref, out_ref):
    plr = plr_ref[...]
    pli = pli_ref[...]
    out_ref[...] = (plr * plr + pli * pli) * tg_ref[...]


def _tc_prod(plint_real, plint_imag, tg):
    B = 8000
    return pl.pallas_call(
        _prod_kernel,
        grid=(E // B,),
        in_specs=[
            pl.BlockSpec((B, A), lambda i: (i, 0)),
            pl.BlockSpec((B, A), lambda i: (i, 0)),
            pl.BlockSpec((B, A), lambda i: (i, 0)),
        ],
        out_specs=pl.BlockSpec((B, A), lambda i: (i, 0)),
        out_shape=jax.ShapeDtypeStruct((E, A), jnp.float32),
    )(plint_real, plint_imag, tg)


def _final_sum_kernel(p0_ref, p1_ref, out_ref):
    out_ref[...] = jnp.sum(p0_ref[...] + p1_ref[...], axis=1, keepdims=True)


def _tc_final_sum(p0, p1):
    B = 6256
    return pl.pallas_call(
        _final_sum_kernel,
        grid=(SEG_PAD // B,),
        in_specs=[
            pl.BlockSpec((B, A), lambda i: (i, 0)),
            pl.BlockSpec((B, A), lambda i: (i, 0)),
        ],
        out_specs=pl.BlockSpec((B, 1), lambda i: (i, 0)),
        out_shape=jax.ShapeDtypeStruct((SEG_PAD, 1), jnp.float32),
    )(p0, p1)


def kernel(p_real, p_imag, plint_real, plint_imag, pldl_real, pldl_imag,
           int_src, int_dst, dlink_src, ulink_src,
           w1a, b1a, w1b, b1b, w2a, b2a, w2b, b2b):
    zstripe = jnp.zeros((STRIPE, A), jnp.float32)

    s2 = _tc_s2_table(p_real, p_imag)

    g = _sc_gather(s2, int_src, 2 * A)
    msg = _tc_edge_mlp(plint_real, plint_imag, g, w2a, b2a, w2b, b2b)
    m0, m1 = _sc_segment_sum(msg, int_dst, zstripe)

    pldl_r3 = pldl_real.reshape(NUM_AP, U, A)
    pldl_i3 = pldl_imag.reshape(NUM_AP, U, A)
    inner = _tc_inner_infer(p_real, p_imag, pldl_r3, pldl_i3)
    uer, uei = _tc_ue_mlp(inner.reshape(NUM_UE, 1), pldl_real, pldl_imag,
                          m0[:NUM_UE], m1[:NUM_UE], w1a, b1a, w1b, b1b)

    outr, outi, t = _tc_normalize(uer.reshape(NUM_AP, U, A),
                                  uei.reshape(NUM_AP, U, A))

    tg = _sc_gather(t, int_src, A)
    prod = _tc_prod(plint_real, plint_imag, tg)
    p0, p1 = _sc_segment_sum(prod, int_dst, zstripe)
    ue2 = _tc_final_sum(p0, p1)[:NUM_UE]

    out = jnp.stack([outr, outi], axis=-1)
    return out, ue2


# speedup vs baseline: 1.8204x; 1.8204x over previous
import functools

import jax
import jax.numpy as jnp
from jax import lax
from jax.experimental import pallas as pl
from jax.experimental.pallas import tpu as pltpu
from jax.experimental.pallas import tpu_sc as plsc

NUM_AP = 5000
U = 10
NUM_UE = 50000
A = 16
E = 1600000
M2 = 16
H = 64

NC = 2
NS = 16
NW = NC * NS
EPT = E // NW
CH = 1000
STRIPE = 3128
SEG_PAD = NS * STRIPE

def _mesh():
    return plsc.VectorSubcoreMesh(core_axis_name="c", subcore_axis_name="s",
                                  num_cores=NC, num_subcores=NS)


def _sc_gather(table, idx, width):
    ch = 1000 if width > A else 2000
    nch = EPT // ch

    @functools.partial(
        pl.kernel,
        out_type=jax.ShapeDtypeStruct((E, width), jnp.float32),
        mesh=_mesh(),
        compiler_params=pltpu.CompilerParams(use_tc_tiling_on_sc=False),
        scratch_types=[
            pltpu.VMEM((2, ch), jnp.int32),
            pltpu.VMEM((2, ch, width), jnp.float32),
            pltpu.SemaphoreType.DMA,
            pltpu.SemaphoreType.DMA,
            pltpu.SemaphoreType.DMA,
            pltpu.SemaphoreType.DMA,
        ],
    )
    def k(table_hbm, idx_hbm, out_hbm, idx_v, rows_v, sg0, sg1, si0, si1):
        wid = lax.axis_index("c") * NS + lax.axis_index("s")
        base = wid * EPT

        sg = [sg0, sg1]
        si = [si0, si1]
        dg = [None] * nch
        di = [None] * nch

        def start_idx(j):
            di[j] = pltpu.async_copy(idx_hbm.at[pl.ds(base + j * ch, ch)],
                                     idx_v.at[j % 2], si[j % 2])

        def start_gather(j):
            dg[j] = pltpu.async_copy(table_hbm.at[idx_v.at[j % 2]],
                                     rows_v.at[j % 2], sg[j % 2])

        start_idx(0)
        di[0].wait()
        start_gather(0)
        if nch > 1:
            start_idx(1)
        for j in range(nch):
            dg[j].wait()
            if j + 2 < nch:
                start_idx(j + 2)
            if j + 1 < nch:
                di[j + 1].wait()
                start_gather(j + 1)
            pltpu.sync_copy(rows_v.at[j % 2], out_hbm.at[pl.ds(base + j * ch, ch)])

    return k(table, idx)


def _sc_segment_sum(rows, idx, zstripe):
    ch = 2000
    nch = EPT // ch

    @functools.partial(
        pl.kernel,
        out_type=[
            jax.ShapeDtypeStruct((SEG_PAD, A), jnp.float32),
            jax.ShapeDtypeStruct((SEG_PAD, A), jnp.float32),
        ],
        mesh=_mesh(),
        compiler_params=pltpu.CompilerParams(use_tc_tiling_on_sc=False),
        scratch_types=[
            pltpu.VMEM_SHARED((SEG_PAD, A), jnp.float32),
            pltpu.VMEM((2, ch), jnp.int32),
            pltpu.VMEM((2, ch, A), jnp.float32),
            pltpu.SemaphoreType.DMA,
            pltpu.SemaphoreType.DMA,
            pltpu.SemaphoreType.DMA,
            pltpu.SemaphoreType.DMA,
        ],
    )
    def k(rows_hbm, idx_hbm, z_hbm, out0, out1, acc, idx_v, rows_v,
          semr0, semr1, semi0, semi1):
        cid = lax.axis_index("c")
        sid = lax.axis_index("s")
        wid = cid * NS + sid
        base = wid * EPT
        pltpu.sync_copy(z_hbm, acc.at[pl.ds(sid * STRIPE, STRIPE)])
        plsc.subcore_barrier()

        semr = [semr0, semr1]
        semi = [semi0, semi1]
        dr = [None] * nch
        di = [None] * nch

        def start(j):
            b = j % 2
            dr[j] = pltpu.async_copy(rows_hbm.at[pl.ds(base + j * ch, ch)],
                                     rows_v.at[b], semr[b])
            di[j] = pltpu.async_copy(idx_hbm.at[pl.ds(base + j * ch, ch)],
                                     idx_v.at[b], semi[b])

        start(0)
        for j in range(nch):
            if j + 1 < nch:
                start(j + 1)
            dr[j].wait()
            di[j].wait()
            pltpu.sync_copy(rows_v.at[j % 2], acc.at[idx_v.at[j % 2]], add=True)

        plsc.subcore_barrier()

        @pl.when(cid == 0)
        def _():
            pltpu.sync_copy(acc.at[pl.ds(sid * STRIPE, STRIPE)],
                            out0.at[pl.ds(sid * STRIPE, STRIPE)])

        @pl.when(cid == 1)
        def _():
            pltpu.sync_copy(acc.at[pl.ds(sid * STRIPE, STRIPE)],
                            out1.at[pl.ds(sid * STRIPE, STRIPE)])

    return k(rows, idx, zstripe)


def _bsrc_kernel(pr_ref, pi_ref, wr_ref, wi_ref, b2a_ref, out_ref):
    s2r = jnp.sum(pr_ref[...], axis=1)
    s2i = jnp.sum(pi_ref[...], axis=1)
    out_ref[...] = (jnp.dot(s2r, wr_ref[...], preferred_element_type=jnp.float32)
                    + jnp.dot(s2i, wi_ref[...], preferred_element_type=jnp.float32)
                    + b2a_ref[...])


def _tc_bsrc_table(p_real, p_imag, w2a, b2a):
    BA = 200
    return pl.pallas_call(
        _bsrc_kernel,
        grid=(NUM_AP // BA,),
        in_specs=[
            pl.BlockSpec((BA, U, A), lambda i: (i, 0, 0)),
            pl.BlockSpec((BA, U, A), lambda i: (i, 0, 0)),
            pl.BlockSpec((A, H), lambda i: (0, 0)),
            pl.BlockSpec((A, H), lambda i: (0, 0)),
            pl.BlockSpec((1, H), lambda i: (0, 0)),
        ],
        out_specs=pl.BlockSpec((BA, H), lambda i: (i, 0)),
        out_shape=jax.ShapeDtypeStruct((NUM_AP, H), jnp.float32),
    )(p_real, p_imag, w2a[A:2 * A], w2a[3 * A:4 * A], b2a.reshape(1, H))


def _edge_mlp_kernel(plr8_ref, pli8_ref, b8_ref, bdr_ref, bdi_ref,
                     bdw2b_ref, b2b8_ref, msg8_ref, q8_ref):
    plr8 = plr8_ref[...]
    pli8 = pli8_ref[...]
    a = (jnp.dot(plr8, bdr_ref[...], preferred_element_type=jnp.float32)
         + jnp.dot(pli8, bdi_ref[...], preferred_element_type=jnp.float32)
         + b8_ref[...])
    h = jnp.maximum(a, 0.0)
    msg8_ref[...] = (jnp.dot(h, bdw2b_ref[...], preferred_element_type=jnp.float32)
                     + b2b8_ref[...])
    q8_ref[...] = plr8 * plr8 + pli8 * pli8


def _tc_edge_mlp(plr8, pli8, b8, w2a, w2b, b2b):
    B8 = 1000
    eye8 = jnp.eye(8, dtype=jnp.float32)
    bdr = jnp.kron(eye8, w2a[0:A])
    bdi = jnp.kron(eye8, w2a[2 * A:3 * A])
    bdw2b = jnp.kron(eye8, w2b)
    b2b8 = jnp.tile(b2b, 8).reshape(1, 8 * M2)
    return pl.pallas_call(
        _edge_mlp_kernel,
        grid=(E // (8 * B8),),
        in_specs=[
            pl.BlockSpec((B8, 128), lambda i: (i, 0)),
            pl.BlockSpec((B8, 128), lambda i: (i, 0)),
            pl.BlockSpec((B8, 8 * H), lambda i: (i, 0)),
            pl.BlockSpec((128, 8 * H), lambda i: (0, 0)),
            pl.BlockSpec((128, 8 * H), lambda i: (0, 0)),
            pl.BlockSpec((8 * H, 128), lambda i: (0, 0)),
            pl.BlockSpec((1, 8 * M2), lambda i: (0, 0)),
        ],
        out_specs=[
            pl.BlockSpec((B8, 128), lambda i: (i, 0)),
            pl.BlockSpec((B8, 128), lambda i: (i, 0)),
        ],
        out_shape=[
            jax.ShapeDtypeStruct((E // 8, 128), jnp.float32),
            jax.ShapeDtypeStruct((E // 8, 128), jnp.float32),
        ],
    )(plr8, pli8, b8, bdr, bdi, bdw2b, b2b8)


def _inner_infer_kernel(pr_ref, pi_ref, qr_ref, qi_ref, out_ref):
    pr = pr_ref[...]
    pi = pi_ref[...]
    qr = qr_ref[...]
    qi = qi_ref[...]
    BA = pr.shape[0]
    vidx = lax.broadcasted_iota(jnp.int32, (BA, U), 1)
    acc = jnp.zeros((BA, U), jnp.float32)
    diag = jnp.zeros((BA, U), jnp.float32)
    for u in range(U):
        pru = pr[:, u:u + 1, :]
        piu = pi[:, u:u + 1, :]
        ir = jnp.sum(pru * qr + piu * qi, axis=-1)
        ii = jnp.sum(pru * qi - piu * qr, axis=-1)
        nrm = ir * ir + ii * ii
        acc = acc + nrm
        diag = diag + jnp.where(vidx == u, nrm, 0.0)
    out_ref[...] = acc - diag


def _tc_inner_infer(p_real, p_imag, pldl_r3, pldl_i3):
    BA = 200
    return pl.pallas_call(
        _inner_infer_kernel,
        grid=(NUM_AP // BA,),
        in_specs=[
            pl.BlockSpec((BA, U, A), lambda i: (i, 0, 0)),
            pl.BlockSpec((BA, U, A), lambda i: (i, 0, 0)),
            pl.BlockSpec((BA, U, A), lambda i: (i, 0, 0)),
            pl.BlockSpec((BA, U, A), lambda i: (i, 0, 0)),
        ],
        out_specs=pl.BlockSpec((BA, U), lambda i: (i, 0)),
        out_shape=jax.ShapeDtypeStruct((NUM_AP, U), jnp.float32),
    )(p_real, p_imag, pldl_r3, pldl_i3)


def _ue_mlp_kernel(inner_ref, plr_ref, pli_ref, m0_ref, m1_ref,
                   w_in_ref, w_plr_ref, w_pli_ref, w_m_ref, b1a_ref,
                   w1b_ref, b1b_ref, uer_ref, uei_ref):
    inner = inner_ref[...]
    m = m0_ref[...] + m1_ref[...]
    h = (inner * w_in_ref[...]
         + jnp.dot(plr_ref[...], w_plr_ref[...], preferred_element_type=jnp.float32)
         + jnp.dot(pli_ref[...], w_pli_ref[...], preferred_element_type=jnp.float32)
         + jnp.dot(m, w_m_ref[...], preferred_element_type=jnp.float32)
         + b1a_ref[...])
    h = jnp.maximum(h, 0.0)
    o = jnp.dot(h, w1b_ref[...], preferred_element_type=jnp.float32) + b1b_ref[...]
    uer_ref[...] = o[:, :A]
    uei_ref[...] = o[:, A:2 * A]


def _tc_ue_mlp(inner, pldl_real, pldl_imag, m0, m1, w1a, b1a, w1b, b1b):
    B = 5000
    return pl.pallas_call(
        _ue_mlp_kernel,
        grid=(NUM_UE // B,),
        in_specs=[
            pl.BlockSpec((B, 1), lambda i: (i, 0)),
            pl.BlockSpec((B, A), lambda i: (i, 0)),
            pl.BlockSpec((B, A), lambda i: (i, 0)),
            pl.BlockSpec((B, M2), lambda i: (i, 0)),
            pl.BlockSpec((B, M2), lambda i: (i, 0)),
            pl.BlockSpec((1, H), lambda i: (0, 0)),
            pl.BlockSpec((A, H), lambda i: (0, 0)),
            pl.BlockSpec((A, H), lambda i: (0, 0)),
            pl.BlockSpec((M2, H), lambda i: (0, 0)),
            pl.BlockSpec((1, H), lambda i: (0, 0)),
            pl.BlockSpec((H, 2 * A), lambda i: (0, 0)),
            pl.BlockSpec((1, 2 * A), lambda i: (0, 0)),
        ],
        out_specs=[
            pl.BlockSpec((B, A), lambda i: (i, 0)),
            pl.BlockSpec((B, A), lambda i: (i, 0)),
        ],
        out_shape=[
            jax.ShapeDtypeStruct((NUM_UE, A), jnp.float32),
            jax.ShapeDtypeStruct((NUM_UE, A), jnp.float32),
        ],
    )(inner, pldl_real, pldl_imag, m0, m1,
      w1a[0:1], w1a[1:1 + A], w1a[1 + A:1 + 2 * A], w1a[1 + 2 * A:1 + 2 * A + M2],
      b1a.reshape(1, H), w1b, b1b.reshape(1, 2 * A))


def _normalize_kernel(r_ref, i_ref, outr_ref, outi_ref, t_ref):
    r = r_ref[...]
    im = i_ref[...]
    mag = jnp.sqrt(r * r + im * im)
    s = jnp.sum(jnp.sum(mag, axis=2, keepdims=True), axis=1, keepdims=True)
    nr = r / s
    ni = im / s
    outr_ref[...] = nr
    outi_ref[...] = ni
    sr = jnp.sum(nr, axis=1)
    si = jnp.sum(ni, axis=1)
    t_ref[...] = sr * sr + si * si


def _tc_normalize(uer3, uei3):
    BA = 200
    return pl.pallas_call(
        _normalize_kernel,
        grid=(NUM_AP // BA,),
        in_specs=[
            pl.BlockSpec((BA, U, A), lambda i: (i, 0, 0)),
            pl.BlockSpec((BA, U, A), lambda i: (i, 0, 0)),
        ],
        out_specs=[
            pl.BlockSpec((BA, U, A), lambda i: (i, 0, 0)),
            pl.BlockSpec((BA, U, A), lambda i: (i, 0, 0)),
            pl.BlockSpec((BA, A), lambda i: (i, 0)),
        ],
        out_shape=[
            jax.ShapeDtypeStruct((NUM_AP, U, A), jnp.float32),
            jax.ShapeDtypeStruct((NUM_AP, U, A), jnp.float32),
            jax.ShapeDtypeStruct((NUM_AP, A), jnp.float32),
        ],
    )(uer3, uei3)


def _prod_kernel(q_ref, tg_ref, out_ref):
    out_ref[...] = q_ref[...] * tg_ref[...]


def _tc_prod(q8, tg8):
    B = 8000
    R = E * A // 128
    return pl.pallas_call(
        _prod_kernel,
        grid=(R // B,),
        in_specs=[
            pl.BlockSpec((B, 128), lambda i: (i, 0)),
            pl.BlockSpec((B, 128), lambda i: (i, 0)),
        ],
        out_specs=pl.BlockSpec((B, 128), lambda i: (i, 0)),
        out_shape=jax.ShapeDtypeStruct((R, 128), jnp.float32),
    )(q8, tg8)


def _final_sum_kernel(p0_ref, p1_ref, out_ref):
    out_ref[...] = jnp.sum(p0_ref[...] + p1_ref[...], axis=1, keepdims=True)


def _tc_final_sum(p0, p1):
    B = 6256
    return pl.pallas_call(
        _final_sum_kernel,
        grid=(SEG_PAD // B,),
        in_specs=[
            pl.BlockSpec((B, A), lambda i: (i, 0)),
            pl.BlockSpec((B, A), lambda i: (i, 0)),
        ],
        out_specs=pl.BlockSpec((B, 1), lambda i: (i, 0)),
        out_shape=jax.ShapeDtypeStruct((SEG_PAD, 1), jnp.float32),
    )(p0, p1)


def kernel(p_real, p_imag, plint_real, plint_imag, pldl_real, pldl_imag,
           int_src, int_dst, dlink_src, ulink_src,
           w1a, b1a, w1b, b1b, w2a, b2a, w2b, b2b):
    zstripe = jnp.zeros((STRIPE, A), jnp.float32)

    bsrc = _tc_bsrc_table(p_real, p_imag, w2a, b2a)

    b = _sc_gather(bsrc, int_src, H)
    msg8, q8 = _tc_edge_mlp(plint_real.reshape(E // 8, 128),
                            plint_imag.reshape(E // 8, 128),
                            b.reshape(E // 8, 8 * H), w2a, w2b, b2b)
    m0, m1 = _sc_segment_sum(msg8.reshape(E, M2), int_dst, zstripe)

    pldl_r3 = pldl_real.reshape(NUM_AP, U, A)
    pldl_i3 = pldl_imag.reshape(NUM_AP, U, A)
    inner = _tc_inner_infer(p_real, p_imag, pldl_r3, pldl_i3)
    uer, uei = _tc_ue_mlp(inner.reshape(NUM_UE, 1), pldl_real, pldl_imag,
                          m0[:NUM_UE], m1[:NUM_UE], w1a, b1a, w1b, b1b)

    outr, outi, t = _tc_normalize(uer.reshape(NUM_AP, U, A),
                                  uei.reshape(NUM_AP, U, A))

    tg = _sc_gather(t, int_src, A)
    prod8 = _tc_prod(q8, tg.reshape(E * A // 128, 128))
    p0, p1 = _sc_segment_sum(prod8.reshape(E, A), int_dst, zstripe)
    ue2 = _tc_final_sum(p0, p1)[:NUM_UE]

    out = jnp.stack([outr, outi], axis=-1)
    return out, ue2


# speedup vs baseline: 1.8570x; 1.0201x over previous
import functools

import jax
import jax.numpy as jnp
from jax import lax
from jax.experimental import pallas as pl
from jax.experimental.pallas import tpu as pltpu
from jax.experimental.pallas import tpu_sc as plsc

NUM_AP = 5000
U = 10
NUM_UE = 50000
A = 16
E = 1600000
M2 = 16
H = 64

NC = 2
NS = 16
NW = NC * NS
EPT = E // NW
CH = 1000
STRIPE = 3128
SEG_PAD = NS * STRIPE

def _mesh():
    return plsc.VectorSubcoreMesh(core_axis_name="c", subcore_axis_name="s",
                                  num_cores=NC, num_subcores=NS)


def _sc_gather(table, idx, width):
    ch = 1000 if width > A else 2000
    nch = EPT // ch

    @functools.partial(
        pl.kernel,
        out_type=jax.ShapeDtypeStruct((E, width), jnp.float32),
        mesh=_mesh(),
        compiler_params=pltpu.CompilerParams(use_tc_tiling_on_sc=False),
        scratch_types=[
            pltpu.VMEM((2, ch), jnp.int32),
            pltpu.VMEM((2, ch, width), jnp.float32),
            pltpu.SemaphoreType.DMA,
            pltpu.SemaphoreType.DMA,
            pltpu.SemaphoreType.DMA,
            pltpu.SemaphoreType.DMA,
        ],
    )
    def k(table_hbm, idx_hbm, out_hbm, idx_v, rows_v, sg0, sg1, si0, si1):
        wid = lax.axis_index("c") * NS + lax.axis_index("s")
        base = wid * EPT

        sg = [sg0, sg1]
        si = [si0, si1]
        dg = [None] * nch
        di = [None] * nch

        def start_idx(j):
            di[j] = pltpu.async_copy(idx_hbm.at[pl.ds(base + j * ch, ch)],
                                     idx_v.at[j % 2], si[j % 2])

        def start_gather(j):
            dg[j] = pltpu.async_copy(table_hbm.at[idx_v.at[j % 2]],
                                     rows_v.at[j % 2], sg[j % 2])

        start_idx(0)
        di[0].wait()
        start_gather(0)
        if nch > 1:
            start_idx(1)
        for j in range(nch):
            dg[j].wait()
            if j + 2 < nch:
                start_idx(j + 2)
            if j + 1 < nch:
                di[j + 1].wait()
                start_gather(j + 1)
            pltpu.sync_copy(rows_v.at[j % 2], out_hbm.at[pl.ds(base + j * ch, ch)])

    return k(table, idx)


def _sc_segment_sum(rows, idx, zstripe):
    ch = 2000
    nch = EPT // ch

    @functools.partial(
        pl.kernel,
        out_type=[
            jax.ShapeDtypeStruct((SEG_PAD, A), jnp.float32),
            jax.ShapeDtypeStruct((SEG_PAD, A), jnp.float32),
        ],
        mesh=_mesh(),
        compiler_params=pltpu.CompilerParams(use_tc_tiling_on_sc=False),
        scratch_types=[
            pltpu.VMEM_SHARED((SEG_PAD, A), jnp.float32),
            pltpu.VMEM((2, ch), jnp.int32),
            pltpu.VMEM((2, ch, A), jnp.float32),
            pltpu.SemaphoreType.DMA,
            pltpu.SemaphoreType.DMA,
            pltpu.SemaphoreType.DMA,
            pltpu.SemaphoreType.DMA,
        ],
    )
    def k(rows_hbm, idx_hbm, z_hbm, out0, out1, acc, idx_v, rows_v,
          semr0, semr1, semi0, semi1):
        cid = lax.axis_index("c")
        sid = lax.axis_index("s")
        wid = cid * NS + sid
        base = wid * EPT
        pltpu.sync_copy(z_hbm, acc.at[pl.ds(sid * STRIPE, STRIPE)])
        plsc.subcore_barrier()

        semr = [semr0, semr1]
        semi = [semi0, semi1]
        dr = [None] * nch
        di = [None] * nch

        def start(j):
            b = j % 2
            dr[j] = pltpu.async_copy(rows_hbm.at[pl.ds(base + j * ch, ch)],
                                     rows_v.at[b], semr[b])
            di[j] = pltpu.async_copy(idx_hbm.at[pl.ds(base + j * ch, ch)],
                                     idx_v.at[b], semi[b])

        start(0)
        for j in range(nch):
            if j + 1 < nch:
                start(j + 1)
            dr[j].wait()
            di[j].wait()
            pltpu.sync_copy(rows_v.at[j % 2], acc.at[idx_v.at[j % 2]], add=True)

        plsc.subcore_barrier()

        @pl.when(cid == 0)
        def _():
            pltpu.sync_copy(acc.at[pl.ds(sid * STRIPE, STRIPE)],
                            out0.at[pl.ds(sid * STRIPE, STRIPE)])

        @pl.when(cid == 1)
        def _():
            pltpu.sync_copy(acc.at[pl.ds(sid * STRIPE, STRIPE)],
                            out1.at[pl.ds(sid * STRIPE, STRIPE)])

    return k(rows, idx, zstripe)


def _bsrc_kernel(pr_ref, pi_ref, wr_ref, wi_ref, b2a_ref, out_ref):
    s2r = jnp.sum(pr_ref[...], axis=1)
    s2i = jnp.sum(pi_ref[...], axis=1)
    out_ref[...] = (jnp.dot(s2r, wr_ref[...], preferred_element_type=jnp.float32)
                    + jnp.dot(s2i, wi_ref[...], preferred_element_type=jnp.float32)
                    + b2a_ref[...])


def _tc_bsrc_table(p_real, p_imag, w2a, b2a):
    BA = 200
    return pl.pallas_call(
        _bsrc_kernel,
        grid=(NUM_AP // BA,),
        in_specs=[
            pl.BlockSpec((BA, U, A), lambda i: (i, 0, 0)),
            pl.BlockSpec((BA, U, A), lambda i: (i, 0, 0)),
            pl.BlockSpec((A, H), lambda i: (0, 0)),
            pl.BlockSpec((A, H), lambda i: (0, 0)),
            pl.BlockSpec((1, H), lambda i: (0, 0)),
        ],
        out_specs=pl.BlockSpec((BA, H), lambda i: (i, 0)),
        out_shape=jax.ShapeDtypeStruct((NUM_AP, H), jnp.float32),
    )(p_real, p_imag, w2a[A:2 * A], w2a[3 * A:4 * A], b2a.reshape(1, H))


def _edge_mlp_kernel(plr8_ref, pli8_ref, b8_ref, bdr_ref, bdi_ref,
                     bdw2b_ref, b2b8_ref, msg8_ref, q8_ref):
    plr8 = plr8_ref[...]
    pli8 = pli8_ref[...]
    a = (jnp.dot(plr8, bdr_ref[...], preferred_element_type=jnp.float32)
         + jnp.dot(pli8, bdi_ref[...], preferred_element_type=jnp.float32)
         + b8_ref[...])
    h = jnp.maximum(a, 0.0)
    msg8_ref[...] = (jnp.dot(h, bdw2b_ref[...], preferred_element_type=jnp.float32)
                     + b2b8_ref[...])
    q8_ref[...] = plr8 * plr8 + pli8 * pli8


def _tc_edge_mlp(plr8, pli8, b8, w2a, w2b, b2b):
    B8 = 2000
    eye8 = jnp.eye(8, dtype=jnp.float32)
    bdr = jnp.kron(eye8, w2a[0:A])
    bdi = jnp.kron(eye8, w2a[2 * A:3 * A])
    bdw2b = jnp.kron(eye8, w2b)
    b2b8 = jnp.tile(b2b, 8).reshape(1, 8 * M2)
    return pl.pallas_call(
        _edge_mlp_kernel,
        grid=(E // (8 * B8),),
        in_specs=[
            pl.BlockSpec((B8, 128), lambda i: (i, 0)),
            pl.BlockSpec((B8, 128), lambda i: (i, 0)),
            pl.BlockSpec((B8, 8 * H), lambda i: (i, 0)),
            pl.BlockSpec((128, 8 * H), lambda i: (0, 0)),
            pl.BlockSpec((128, 8 * H), lambda i: (0, 0)),
            pl.BlockSpec((8 * H, 128), lambda i: (0, 0)),
            pl.BlockSpec((1, 8 * M2), lambda i: (0, 0)),
        ],
        out_specs=[
            pl.BlockSpec((B8, 128), lambda i: (i, 0)),
            pl.BlockSpec((B8, 128), lambda i: (i, 0)),
        ],
        out_shape=[
            jax.ShapeDtypeStruct((E // 8, 128), jnp.float32),
            jax.ShapeDtypeStruct((E // 8, 128), jnp.float32),
        ],
    )(plr8, pli8, b8, bdr, bdi, bdw2b, b2b8)


def _inner_infer_kernel(pr_ref, pi_ref, qr_ref, qi_ref, out_ref):
    pr = pr_ref[...]
    pi = pi_ref[...]
    qr = qr_ref[...]
    qi = qi_ref[...]
    BA = pr.shape[0]
    vidx = lax.broadcasted_iota(jnp.int32, (BA, U), 1)
    acc = jnp.zeros((BA, U), jnp.float32)
    diag = jnp.zeros((BA, U), jnp.float32)
    for u in range(U):
        pru = pr[:, u:u + 1, :]
        piu = pi[:, u:u + 1, :]
        ir = jnp.sum(pru * qr + piu * qi, axis=-1)
        ii = jnp.sum(pru * qi - piu * qr, axis=-1)
        nrm = ir * ir + ii * ii
        acc = acc + nrm
        diag = diag + jnp.where(vidx == u, nrm, 0.0)
    out_ref[...] = acc - diag


def _tc_inner_infer(p_real, p_imag, pldl_r3, pldl_i3):
    BA = 200
    return pl.pallas_call(
        _inner_infer_kernel,
        grid=(NUM_AP // BA,),
        in_specs=[
            pl.BlockSpec((BA, U, A), lambda i: (i, 0, 0)),
            pl.BlockSpec((BA, U, A), lambda i: (i, 0, 0)),
            pl.BlockSpec((BA, U, A), lambda i: (i, 0, 0)),
            pl.BlockSpec((BA, U, A), lambda i: (i, 0, 0)),
        ],
        out_specs=pl.BlockSpec((BA, U), lambda i: (i, 0)),
        out_shape=jax.ShapeDtypeStruct((NUM_AP, U), jnp.float32),
    )(p_real, p_imag, pldl_r3, pldl_i3)


def _ue_mlp_kernel(inner_ref, plr_ref, pli_ref, m0_ref, m1_ref,
                   w_in_ref, w_plr_ref, w_pli_ref, w_m_ref, b1a_ref,
                   w1b_ref, b1b_ref, uer_ref, uei_ref):
    inner = inner_ref[...]
    m = m0_ref[...] + m1_ref[...]
    h = (inner * w_in_ref[...]
         + jnp.dot(plr_ref[...], w_plr_ref[...], preferred_element_type=jnp.float32)
         + jnp.dot(pli_ref[...], w_pli_ref[...], preferred_element_type=jnp.float32)
         + jnp.dot(m, w_m_ref[...], preferred_element_type=jnp.float32)
         + b1a_ref[...])
    h = jnp.maximum(h, 0.0)
    o = jnp.dot(h, w1b_ref[...], preferred_element_type=jnp.float32) + b1b_ref[...]
    uer_ref[...] = o[:, :A]
    uei_ref[...] = o[:, A:2 * A]


def _tc_ue_mlp(inner, pldl_real, pldl_imag, m0, m1, w1a, b1a, w1b, b1b):
    B = 5000
    return pl.pallas_call(
        _ue_mlp_kernel,
        grid=(NUM_UE // B,),
        in_specs=[
            pl.BlockSpec((B, 1), lambda i: (i, 0)),
            pl.BlockSpec((B, A), lambda i: (i, 0)),
            pl.BlockSpec((B, A), lambda i: (i, 0)),
            pl.BlockSpec((B, M2), lambda i: (i, 0)),
            pl.BlockSpec((B, M2), lambda i: (i, 0)),
            pl.BlockSpec((1, H), lambda i: (0, 0)),
            pl.BlockSpec((A, H), lambda i: (0, 0)),
            pl.BlockSpec((A, H), lambda i: (0, 0)),
            pl.BlockSpec((M2, H), lambda i: (0, 0)),
            pl.BlockSpec((1, H), lambda i: (0, 0)),
            pl.BlockSpec((H, 2 * A), lambda i: (0, 0)),
            pl.BlockSpec((1, 2 * A), lambda i: (0, 0)),
        ],
        out_specs=[
            pl.BlockSpec((B, A), lambda i: (i, 0)),
            pl.BlockSpec((B, A), lambda i: (i, 0)),
        ],
        out_shape=[
            jax.ShapeDtypeStruct((NUM_UE, A), jnp.float32),
            jax.ShapeDtypeStruct((NUM_UE, A), jnp.float32),
        ],
    )(inner, pldl_real, pldl_imag, m0, m1,
      w1a[0:1], w1a[1:1 + A], w1a[1 + A:1 + 2 * A], w1a[1 + 2 * A:1 + 2 * A + M2],
      b1a.reshape(1, H), w1b, b1b.reshape(1, 2 * A))


def _normalize_kernel(r_ref, i_ref, outr_ref, outi_ref, t_ref):
    r = r_ref[...]
    im = i_ref[...]
    mag = jnp.sqrt(r * r + im * im)
    s = jnp.sum(jnp.sum(mag, axis=2, keepdims=True), axis=1, keepdims=True)
    nr = r / s
    ni = im / s
    outr_ref[...] = nr
    outi_ref[...] = ni
    sr = jnp.sum(nr, axis=1)
    si = jnp.sum(ni, axis=1)
    t_ref[...] = sr * sr + si * si


def _tc_normalize(uer3, uei3):
    BA = 200
    return pl.pallas_call(
        _normalize_kernel,
        grid=(NUM_AP // BA,),
        in_specs=[
            pl.BlockSpec((BA, U, A), lambda i: (i, 0, 0)),
            pl.BlockSpec((BA, U, A), lambda i: (i, 0, 0)),
        ],
        out_specs=[
            pl.BlockSpec((BA, U, A), lambda i: (i, 0, 0)),
            pl.BlockSpec((BA, U, A), lambda i: (i, 0, 0)),
            pl.BlockSpec((BA, A), lambda i: (i, 0)),
        ],
        out_shape=[
            jax.ShapeDtypeStruct((NUM_AP, U, A), jnp.float32),
            jax.ShapeDtypeStruct((NUM_AP, U, A), jnp.float32),
            jax.ShapeDtypeStruct((NUM_AP, A), jnp.float32),
        ],
    )(uer3, uei3)


def _prod_kernel(q_ref, tg_ref, out_ref):
    out_ref[...] = q_ref[...] * tg_ref[...]


def _tc_prod(q8, tg8):
    B = 8000
    R = E * A // 128
    return pl.pallas_call(
        _prod_kernel,
        grid=(R // B,),
        in_specs=[
            pl.BlockSpec((B, 128), lambda i: (i, 0)),
            pl.BlockSpec((B, 128), lambda i: (i, 0)),
        ],
        out_specs=pl.BlockSpec((B, 128), lambda i: (i, 0)),
        out_shape=jax.ShapeDtypeStruct((R, 128), jnp.float32),
    )(q8, tg8)


def _final_sum_kernel(p0_ref, p1_ref, out_ref):
    out_ref[...] = jnp.sum(p0_ref[...] + p1_ref[...], axis=1, keepdims=True)


def _tc_final_sum(p0, p1):
    B = 6256
    return pl.pallas_call(
        _final_sum_kernel,
        grid=(SEG_PAD // B,),
        in_specs=[
            pl.BlockSpec((B, A), lambda i: (i, 0)),
            pl.BlockSpec((B, A), lambda i: (i, 0)),
        ],
        out_specs=pl.BlockSpec((B, 1), lambda i: (i, 0)),
        out_shape=jax.ShapeDtypeStruct((SEG_PAD, 1), jnp.float32),
    )(p0, p1)


def kernel(p_real, p_imag, plint_real, plint_imag, pldl_real, pldl_imag,
           int_src, int_dst, dlink_src, ulink_src,
           w1a, b1a, w1b, b1b, w2a, b2a, w2b, b2b):
    zstripe = jnp.zeros((STRIPE, A), jnp.float32)
    plr8 = plint_real.reshape(E // 8, 128)
    pli8 = plint_imag.reshape(E // 8, 128)

    bsrc = _tc_bsrc_table(p_real, p_imag, w2a, b2a)

    b = _sc_gather(bsrc, int_src, H)
    msg8, q8 = _tc_edge_mlp(plr8, pli8,
                            b.reshape(E // 8, 8 * H), w2a, w2b, b2b)
    m0, m1 = _sc_segment_sum(msg8.reshape(E, M2), int_dst, zstripe)

    pldl_r3 = pldl_real.reshape(NUM_AP, U, A)
    pldl_i3 = pldl_imag.reshape(NUM_AP, U, A)
    inner = _tc_inner_infer(p_real, p_imag, pldl_r3, pldl_i3)
    uer, uei = _tc_ue_mlp(inner.reshape(NUM_UE, 1), pldl_real, pldl_imag,
                          m0[:NUM_UE], m1[:NUM_UE], w1a, b1a, w1b, b1b)

    outr, outi, t = _tc_normalize(uer.reshape(NUM_AP, U, A),
                                  uei.reshape(NUM_AP, U, A))

    tg = _sc_gather(t, int_src, A)
    prod8 = _tc_prod(q8, tg.reshape(E * A // 128, 128))
    p0, p1 = _sc_segment_sum(prod8.reshape(E, A), int_dst, zstripe)
    ue2 = _tc_final_sum(p0, p1)[:NUM_UE]

    out = jnp.stack([outr, outi], axis=-1)
    return out, ue2
